# baseline (speedup 1.0000x reference)
import functools

import jax
import jax.numpy as jnp
from jax import lax
from jax.experimental import pallas as pl
from jax.experimental.pallas import tpu as pltpu
from jax.experimental.pallas import tpu_sc as plsc

N_ROWS = 16384
DIM = 128
TC_BLOCK = 1024


@functools.partial(
    pl.kernel,
    mesh=plsc.VectorSubcoreMesh(core_axis_name="c", subcore_axis_name="s"),
    out_type=jax.ShapeDtypeStruct((N_ROWS,), jnp.float32),
    scratch_types=[
        pltpu.VMEM((16, DIM), jnp.float32),
        pltpu.VMEM((16,), jnp.float32),
    ],
    compiler_params=pltpu.CompilerParams(needs_layout_passes=False),
)
def _probe_sc(a_hbm, b_hbm, out_hbm, buf, obuf):
    wid = lax.axis_index("s") * 2 + lax.axis_index("c")
    base = wid * (N_ROWS // 32)
    pltpu.sync_copy(a_hbm.at[pl.ds(base, 16)], buf)
    obuf[...] = buf[0, pl.ds(0, 16)] * 0.0
    pltpu.sync_copy(obuf, out_hbm.at[pl.ds(base, 16)])


def _tc_body(a_ref, b_ref, o_ref):
    o_ref[...] = jnp.sum(a_ref[...] * b_ref[...], axis=1)


_tc_call = pl.pallas_call(
    _tc_body,
    grid=(N_ROWS // TC_BLOCK,),
    in_specs=[
        pl.BlockSpec((TC_BLOCK, DIM), lambda i: (i, 0)),
        pl.BlockSpec((TC_BLOCK, DIM), lambda i: (i, 0)),
    ],
    out_specs=pl.BlockSpec((TC_BLOCK,), lambda i: (i,)),
    out_shape=jax.ShapeDtypeStruct((N_ROWS,), jnp.float32),
)


def kernel(user_emb, items_emb):
    sc_out = _probe_sc(user_emb, items_emb)
    tc_out = _tc_call(user_emb, items_emb)
    return tc_out + sc_out * 0.0
